# baseline (speedup 1.0000x reference)
import functools

import jax
import jax.numpy as jnp
from jax import lax
from jax.experimental import pallas as pl
from jax.experimental.pallas import tpu as pltpu
from jax.experimental.pallas import tpu_sc as plsc

NC = 2
NS = 16
LANES = 16

SB = 800
GW = 128
RING = 2048


def _spmd_sharding():
    mesh = jax.make_mesh((1,), ("_m",))
    return jax.sharding.NamedSharding(mesh, jax.sharding.PartitionSpec())


def _seg_body(x_ref, src_ref, dst_ref, typ_ref, ef_ref,
              a_out, q_out, c_out,
              srcv, dstv, typv, gbuf, dbuf, ebuf,
              ga, da, ea, gb, db, eb, cidx,
              rows_a, rows_b, ef_a, ef_b, cntv,
              a_sh, q_sh, c_sh, sem_a, sem_b, sem_sa, sem_sb, sem_p,
              *, n_pad, n_edges, n_rel, d_in, d_e):
    c = lax.axis_index("c")
    sb = lax.axis_index("s")
    ep = n_edges // NS
    nbatch = ep // SB
    ngrp = SB // LANES
    rows_per_tile = n_pad // NS
    crows = n_pad // 128
    ar16 = jax.lax.iota(jnp.int32, LANES)
    ones_f = jnp.ones((LANES,), jnp.float32)
    zeros_i = jnp.zeros((LANES,), jnp.int32)
    dump_i = jnp.full((LANES,), n_pad - 1, jnp.int32)

    def zero_rows():
        def zb(i, t):
            for g in range(d_in // (2 * LANES)):
                rows_a[i, pl.ds(g * 2 * LANES, 2 * LANES)] = jnp.zeros(
                    (2 * LANES,), jnp.bfloat16)
            return t
        lax.fori_loop(0, GW, zb, 0)

    def zero_cntv():
        def zb(i, t):
            for g in range(128 // LANES):
                cntv[i, pl.ds(g * LANES, LANES)] = jnp.zeros((LANES,), jnp.float32)
            return t
        lax.fori_loop(0, crows, zb, 0)

    def zero_efrows():
        def zb(i, t):
            ef_a[pl.ds(2 * i, 2), pl.ds(0, LANES)] = jnp.zeros(
                (2, LANES), jnp.bfloat16)
            return t
        lax.fori_loop(0, GW // 2, zb, 0)

    def zero_shared():
        for j in range(rows_per_tile // GW):
            pltpu.sync_copy(rows_a, a_sh.at[pl.ds(sb * rows_per_tile + j * GW, GW)])
        for j in range(rows_per_tile // GW):
            pltpu.sync_copy(ef_a, q_sh.at[pl.ds(sb * rows_per_tile + j * GW, GW)])
        @pl.when(sb < crows // 8)
        def _():
            pltpu.sync_copy(cntv.at[pl.ds(0, 8)], c_sh.at[pl.ds(sb * 8, 8)])

    zero_rows()
    zero_efrows()
    zero_cntv()
    zero_shared()
    for g in range(crows // LANES):
        cidx[pl.ds(g * LANES, LANES)] = jnp.full(
            (LANES,), g * LANES, jnp.int32) + ar16
    plsc.subcore_barrier()

    RM = RING - 1
    PW = 2 * GW

    for p in range(n_rel // NC):
        r = c * (n_rel // NC) + p
        rv = jnp.full((LANES,), r, jnp.int32)

        def prep(base, gx, dx, ex):
            for g in range(GW // LANES):
                sl = pl.ds(base + g * LANES, LANES)
                osl = pl.ds(g * LANES, LANES)
                gx[osl] = gbuf[sl]
                dx[osl] = dbuf[sl]
                ex[osl] = ebuf[sl]

        def wait_scatters():
            pltpu.make_async_copy(rows_a, a_sh.at[da], sem_sa).wait()
            pltpu.make_async_copy(ef_a, q_sh.at[da], sem_sa).wait()
            pltpu.make_async_copy(rows_b, a_sh.at[db], sem_sb).wait()
            pltpu.make_async_copy(ef_b, q_sh.at[db], sem_sb).wait()

        def pair(j, carry):
            dr, pend = carry
            @pl.when(pend != 0)
            def _():
                wait_scatters()
            prep(dr & RM, ga, da, ea)
            a1 = pltpu.async_copy(x_ref.at[ga], rows_a, sem_a)
            a2 = pltpu.async_copy(ef_ref.at[ea], ef_a, sem_a)
            prep((dr + GW) & RM, gb, db, eb)
            b1 = pltpu.async_copy(x_ref.at[gb], rows_b, sem_b)
            b2 = pltpu.async_copy(ef_ref.at[eb], ef_b, sem_b)
            a1.wait()
            a2.wait()
            pltpu.async_copy(rows_a, a_sh.at[da], sem_sa, add=True)
            pltpu.async_copy(ef_a, q_sh.at[da], sem_sa, add=True)
            b1.wait()
            b2.wait()
            pltpu.async_copy(rows_b, a_sh.at[db], sem_sb, add=True)
            pltpu.async_copy(ef_b, q_sh.at[db], sem_sb, add=True)
            return (dr + PW, jnp.int32(1))

        off0 = sb * ep
        pltpu.async_copy(src_ref.at[pl.ds(off0, SB)], srcv.at[0], sem_p)
        pltpu.async_copy(dst_ref.at[pl.ds(off0, SB)], dstv.at[0], sem_p)
        pltpu.async_copy(typ_ref.at[pl.ds(off0, SB)], typv.at[0], sem_p)

        def batch_body(jb, carry):
            k, dr, pend = carry
            par = jb & 1
            off = sb * ep + jb * SB
            pltpu.make_async_copy(src_ref.at[pl.ds(off0, SB)],
                                  srcv.at[par], sem_p).wait()
            pltpu.make_async_copy(src_ref.at[pl.ds(off0, SB)],
                                  dstv.at[par], sem_p).wait()
            pltpu.make_async_copy(src_ref.at[pl.ds(off0, SB)],
                                  typv.at[par], sem_p).wait()

            @pl.when(jb + 1 < nbatch)
            def _():
                off2 = off + SB
                nxt = 1 - par
                pltpu.async_copy(src_ref.at[pl.ds(off2, SB)],
                                 srcv.at[nxt], sem_p)
                pltpu.async_copy(dst_ref.at[pl.ds(off2, SB)],
                                 dstv.at[nxt], sem_p)
                pltpu.async_copy(typ_ref.at[pl.ds(off2, SB)],
                                 typv.at[nxt], sem_p)

            def grp(g, k):
                sl = pl.ds(g * LANES, LANES)
                s16 = srcv[par, sl]
                d16 = dstv[par, sl]
                t16 = typv[par, sl]
                m = t16 == rv
                mi = m.astype(jnp.int32)
                pos = (jnp.full((LANES,), k - 1, jnp.int32)
                       + plsc.cumsum(mi)) & jnp.full((LANES,), RM, jnp.int32)
                plsc.store_scatter(gbuf, [pos], s16, mask=m)
                plsc.store_scatter(dbuf, [pos], d16, mask=m)
                eid = jnp.full((LANES,), off + g * LANES, jnp.int32) + ar16
                plsc.store_scatter(ebuf, [pos], eid, mask=m)
                plsc.addupdate_scatter(cntv, [d16 >> 7, d16 & 127], ones_f,
                                       mask=m)
                return k + jnp.sum(mi)

            k = lax.fori_loop(0, ngrp, grp, k)
            dr, pend = lax.fori_loop(0, (k - dr) // PW, pair, (dr, pend))
            return (k, dr, pend)

        k, dr, pend = lax.fori_loop(0, nbatch, batch_body,
                                    (jnp.int32(0), jnp.int32(0),
                                     jnp.int32(0)))

        kpad = ((k + PW - 1) // PW) * PW
        for g in range(PW // LANES):
            idxg = (jnp.full((LANES,), k + g * LANES, jnp.int32) + ar16) \
                & jnp.full((LANES,), RM, jnp.int32)
            plsc.store_scatter(gbuf, [idxg], zeros_i)
            plsc.store_scatter(dbuf, [idxg], dump_i)
            plsc.store_scatter(ebuf, [idxg], zeros_i)
        dr, pend = lax.fori_loop(0, (kpad - dr) // PW, pair, (dr, pend))
        @pl.when(pend != 0)
        def _():
            wait_scatters()

        pltpu.sync_copy(cntv, c_sh.at[cidx], add=True)
        plsc.subcore_barrier()

        pltpu.sync_copy(a_sh.at[pl.ds(sb * rows_per_tile, rows_per_tile)],
                        a_out.at[r, pl.ds(sb * rows_per_tile, rows_per_tile)])
        pltpu.sync_copy(q_sh.at[pl.ds(sb * rows_per_tile, rows_per_tile)],
                        q_out.at[r, pl.ds(sb * rows_per_tile, rows_per_tile)])
        @pl.when(sb < crows // 8)
        def _():
            pltpu.sync_copy(c_sh.at[pl.ds(sb * 8, 8)],
                            c_out.at[r, pl.ds(sb * 8, 8)])
        if p + 1 < n_rel // NC:
            zero_rows()
            zero_efrows()
            zero_cntv()
            zero_shared()
            plsc.subcore_barrier()


def _final_body(x_ref, a_ref, q_ref, c_ref, wa_ref, wb_ref, we_ref, b_ref,
                o_ref, *, n_rel):
    xv = x_ref[...].astype(jnp.bfloat16)
    av = a_ref[...]
    qv = q_ref[...]
    cnt = c_ref[...]
    wav = wa_ref[...]
    wbv = wb_ref[...]
    wbv16 = wbv.astype(jnp.bfloat16)
    wev = we_ref[...]
    bv = b_ref[...]
    s = 1.0 / jnp.maximum(cnt, 1.0)
    rho = jnp.where(cnt > 0.0, 1.0, 0.0).astype(jnp.float32)
    acc = jnp.zeros(o_ref.shape, jnp.float32)
    for r in range(n_rel):
        gw = jnp.dot(av[r], wbv16[r], preferred_element_type=jnp.float32)
        fw = jnp.dot(qv[r], wev[r].astype(jnp.bfloat16),
                     preferred_element_type=jnp.float32)
        zw = jnp.dot(xv, (wav[r] - wbv[r]).astype(jnp.bfloat16),
                     preferred_element_type=jnp.float32)
        acc = acc + s[r][:, None] * (gw + fw) \
                  + rho[r][:, None] * (zw + bv[r][None, :])
    o_ref[...] = acc * (1.0 / n_rel)


def kernel(x, edge_index, edge_type, edge_features, W, b):
    n, d_in = x.shape
    e = edge_index.shape[1]
    n_rel, d_tot, d_out = W.shape
    d_e = d_tot - 2 * d_in
    step = NS * GW
    n_pad = ((n + step - 1) // step) * step

    x_p = jnp.pad(x, ((0, n_pad - n), (0, 0)))
    src = edge_index[0]
    dst = edge_index[1]
    wa = W[:, :d_in, :]
    wb = W[:, d_in:2 * d_in, :]
    we = W[:, 2 * d_in:, :]

    seg = _build_seg(n_pad, e, n_rel, d_in, d_e)
    sh = _spmd_sharding()
    seg_sh = jax.jit(seg, in_shardings=(sh,) * 5, out_shardings=[sh, sh, sh])
    x_sc = x_p.astype(jnp.bfloat16)
    ef_sc = edge_features.astype(jnp.bfloat16)
    agg, q_agg, cnt2d = seg_sh(x_sc, src, dst, edge_type, ef_sc)

    cnt = cnt2d.reshape(n_rel, n_pad)

    bn = 1024
    out_p = pl.pallas_call(
        functools.partial(_final_body, n_rel=n_rel),
        grid=(n_pad // bn,),
        in_specs=[
            pl.BlockSpec((bn, d_in), lambda i: (i, 0)),
            pl.BlockSpec((n_rel, bn, d_in), lambda i: (0, i, 0)),
            pl.BlockSpec((n_rel, bn, d_e), lambda i: (0, i, 0)),
            pl.BlockSpec((n_rel, bn), lambda i: (0, i)),
            pl.BlockSpec((n_rel, d_in, d_out), lambda i: (0, 0, 0)),
            pl.BlockSpec((n_rel, d_in, d_out), lambda i: (0, 0, 0)),
            pl.BlockSpec((n_rel, d_e, d_out), lambda i: (0, 0, 0)),
            pl.BlockSpec((n_rel, d_out), lambda i: (0, 0)),
        ],
        out_specs=pl.BlockSpec((bn, d_out), lambda i: (i, 0)),
        out_shape=jax.ShapeDtypeStruct((n_pad, d_out), jnp.float32),
    )(x_p, agg, q_agg, cnt, wa, wb, we, b)

    return out_p[:n]


def _build_seg(n_pad, e, n_rel, d_in, d_e):
    mesh = plsc.VectorSubcoreMesh(core_axis_name="c", subcore_axis_name="s")
    return pl.kernel(
        functools.partial(_seg_body, n_pad=n_pad, n_edges=e, n_rel=n_rel,
                          d_in=d_in, d_e=d_e),
        out_type=[
            jax.ShapeDtypeStruct((n_rel, n_pad, d_in), jnp.bfloat16),
            jax.ShapeDtypeStruct((n_rel, n_pad, d_e), jnp.bfloat16),
            jax.ShapeDtypeStruct((n_rel, n_pad // 128, 128), jnp.float32),
        ],
        mesh=mesh,
        scratch_types=[
            pltpu.VMEM((2, SB), jnp.int32),
            pltpu.VMEM((2, SB), jnp.int32),
            pltpu.VMEM((2, SB), jnp.int32),
            pltpu.VMEM((RING,), jnp.int32),
            pltpu.VMEM((RING,), jnp.int32),
            pltpu.VMEM((RING,), jnp.int32),
            pltpu.VMEM((GW,), jnp.int32),
            pltpu.VMEM((GW,), jnp.int32),
            pltpu.VMEM((GW,), jnp.int32),
            pltpu.VMEM((GW,), jnp.int32),
            pltpu.VMEM((GW,), jnp.int32),
            pltpu.VMEM((GW,), jnp.int32),
            pltpu.VMEM((n_pad // 128,), jnp.int32),
            pltpu.VMEM((GW, d_in), jnp.bfloat16),
            pltpu.VMEM((GW, d_in), jnp.bfloat16),
            pltpu.VMEM((GW, d_e), jnp.bfloat16),
            pltpu.VMEM((GW, d_e), jnp.bfloat16),
            pltpu.VMEM((n_pad // 128, 128), jnp.float32),
            pltpu.VMEM_SHARED((n_pad, d_in), jnp.bfloat16),
            pltpu.VMEM_SHARED((n_pad, d_e), jnp.bfloat16),
            pltpu.VMEM_SHARED((n_pad // 128, 128), jnp.float32),
            pltpu.SemaphoreType.DMA,
            pltpu.SemaphoreType.DMA,
            pltpu.SemaphoreType.DMA,
            pltpu.SemaphoreType.DMA,
            pltpu.SemaphoreType.DMA,
        ],
        compiler_params=pltpu.CompilerParams(needs_layout_passes=False,
                                             use_tc_tiling_on_sc=False),
    )


# speedup vs baseline: 1.0467x; 1.0467x over previous
import functools

import jax
import jax.numpy as jnp
from jax import lax
from jax.experimental import pallas as pl
from jax.experimental.pallas import tpu as pltpu
from jax.experimental.pallas import tpu_sc as plsc

NC = 2
NS = 16
LANES = 16

SB = 800
GW = 128
RING = 2048


def _spmd_sharding():
    mesh = jax.make_mesh((1,), ("_m",))
    return jax.sharding.NamedSharding(mesh, jax.sharding.PartitionSpec())


def _seg_body(x_ref, src_ref, dst_ref, typ_ref, ef_ref,
              a_out, q_out, c_out,
              srcv, dstv, typv, gbuf, dbuf, ebuf,
              ga, da, ea, gb, db, eb, cidx,
              rows_a, rows_b, ef_a, ef_b, cntv,
              a_sh, q_sh, c_sh, sem_a, sem_b, sem_sa, sem_sb, sem_p,
              *, n_pad, n_edges, n_rel, d_in, d_e):
    c = lax.axis_index("c")
    sb = lax.axis_index("s")
    ep = n_edges // NS
    nbatch = ep // SB
    ngrp = SB // LANES
    rows_per_tile = n_pad // NS
    crows = n_pad // 128
    ar16 = jax.lax.iota(jnp.int32, LANES)
    ones_f = jnp.ones((LANES,), jnp.float32)
    zeros_i = jnp.zeros((LANES,), jnp.int32)
    dump_i = jnp.full((LANES,), n_pad - 1, jnp.int32)

    def zero_rows():
        def zb(i, t):
            for g in range(d_in // (2 * LANES)):
                rows_a[i, pl.ds(g * 2 * LANES, 2 * LANES)] = jnp.zeros(
                    (2 * LANES,), jnp.bfloat16)
            return t
        lax.fori_loop(0, GW, zb, 0)

    def zero_cntv():
        def zb(i, t):
            for g in range(128 // LANES):
                cntv[i, pl.ds(g * LANES, LANES)] = jnp.zeros((LANES,), jnp.float32)
            return t
        lax.fori_loop(0, crows, zb, 0)

    def zero_efrows():
        def zb(i, t):
            ef_a[i, pl.ds(0, LANES)] = jnp.zeros((LANES,), jnp.float32)
            return t
        lax.fori_loop(0, GW, zb, 0)

    def zero_shared():
        for j in range(rows_per_tile // GW):
            pltpu.sync_copy(rows_a, a_sh.at[pl.ds(sb * rows_per_tile + j * GW, GW)])
        for j in range(rows_per_tile // GW):
            pltpu.sync_copy(ef_a, q_sh.at[pl.ds(sb * rows_per_tile + j * GW, GW)])
        @pl.when(sb < crows // 8)
        def _():
            pltpu.sync_copy(cntv.at[pl.ds(0, 8)], c_sh.at[pl.ds(sb * 8, 8)])

    zero_rows()
    zero_efrows()
    zero_cntv()
    zero_shared()
    for g in range(crows // LANES):
        cidx[pl.ds(g * LANES, LANES)] = jnp.full(
            (LANES,), g * LANES, jnp.int32) + ar16
    plsc.subcore_barrier()

    RM = RING - 1
    PW = 2 * GW

    for p in range(n_rel // NC):
        r = c * (n_rel // NC) + p
        rv = jnp.full((LANES,), r, jnp.int32)

        def prep(base, gx, dx, ex):
            for g in range(GW // LANES):
                sl = pl.ds(base + g * LANES, LANES)
                osl = pl.ds(g * LANES, LANES)
                gx[osl] = gbuf[sl]
                dx[osl] = dbuf[sl]
                ex[osl] = ebuf[sl]

        def wait_scatters():
            pltpu.make_async_copy(rows_a, a_sh.at[da], sem_sa).wait()
            pltpu.make_async_copy(ef_a, q_sh.at[da], sem_sa).wait()
            pltpu.make_async_copy(rows_b, a_sh.at[db], sem_sb).wait()
            pltpu.make_async_copy(ef_b, q_sh.at[db], sem_sb).wait()

        def pair(j, carry):
            dr, pend = carry
            @pl.when(pend != 0)
            def _():
                wait_scatters()
            prep(dr & RM, ga, da, ea)
            a1 = pltpu.async_copy(x_ref.at[ga], rows_a, sem_a)
            a2 = pltpu.async_copy(ef_ref.at[ea], ef_a, sem_a)
            prep((dr + GW) & RM, gb, db, eb)
            b1 = pltpu.async_copy(x_ref.at[gb], rows_b, sem_b)
            b2 = pltpu.async_copy(ef_ref.at[eb], ef_b, sem_b)
            a1.wait()
            a2.wait()
            pltpu.async_copy(rows_a, a_sh.at[da], sem_sa, add=True)
            pltpu.async_copy(ef_a, q_sh.at[da], sem_sa, add=True)
            b1.wait()
            b2.wait()
            pltpu.async_copy(rows_b, a_sh.at[db], sem_sb, add=True)
            pltpu.async_copy(ef_b, q_sh.at[db], sem_sb, add=True)
            return (dr + PW, jnp.int32(1))

        off0 = sb * ep
        pltpu.async_copy(src_ref.at[pl.ds(off0, SB)], srcv.at[0], sem_p)
        pltpu.async_copy(dst_ref.at[pl.ds(off0, SB)], dstv.at[0], sem_p)
        pltpu.async_copy(typ_ref.at[pl.ds(off0, SB)], typv.at[0], sem_p)

        def batch_body(jb, carry):
            k, dr, pend = carry
            par = jb & 1
            off = sb * ep + jb * SB
            pltpu.make_async_copy(src_ref.at[pl.ds(off0, SB)],
                                  srcv.at[par], sem_p).wait()
            pltpu.make_async_copy(src_ref.at[pl.ds(off0, SB)],
                                  dstv.at[par], sem_p).wait()
            pltpu.make_async_copy(src_ref.at[pl.ds(off0, SB)],
                                  typv.at[par], sem_p).wait()

            @pl.when(jb + 1 < nbatch)
            def _():
                off2 = off + SB
                nxt = 1 - par
                pltpu.async_copy(src_ref.at[pl.ds(off2, SB)],
                                 srcv.at[nxt], sem_p)
                pltpu.async_copy(dst_ref.at[pl.ds(off2, SB)],
                                 dstv.at[nxt], sem_p)
                pltpu.async_copy(typ_ref.at[pl.ds(off2, SB)],
                                 typv.at[nxt], sem_p)

            def grp(g, k):
                sl = pl.ds(g * LANES, LANES)
                s16 = srcv[par, sl]
                d16 = dstv[par, sl]
                t16 = typv[par, sl]
                m = t16 == rv
                mi = m.astype(jnp.int32)
                pos = (jnp.full((LANES,), k - 1, jnp.int32)
                       + plsc.cumsum(mi)) & jnp.full((LANES,), RM, jnp.int32)
                plsc.store_scatter(gbuf, [pos], s16, mask=m)
                plsc.store_scatter(dbuf, [pos], d16, mask=m)
                eid = jnp.full((LANES,), off + g * LANES, jnp.int32) + ar16
                plsc.store_scatter(ebuf, [pos], eid, mask=m)
                plsc.addupdate_scatter(cntv, [d16 >> 7, d16 & 127], ones_f,
                                       mask=m)
                return k + jnp.sum(mi)

            k = lax.fori_loop(0, ngrp, grp, k)
            dr, pend = lax.fori_loop(0, (k - dr) // PW, pair, (dr, pend))
            return (k, dr, pend)

        k, dr, pend = lax.fori_loop(0, nbatch, batch_body,
                                    (jnp.int32(0), jnp.int32(0),
                                     jnp.int32(0)))

        kpad = ((k + PW - 1) // PW) * PW
        for g in range(PW // LANES):
            idxg = (jnp.full((LANES,), k + g * LANES, jnp.int32) + ar16) \
                & jnp.full((LANES,), RM, jnp.int32)
            plsc.store_scatter(gbuf, [idxg], zeros_i)
            plsc.store_scatter(dbuf, [idxg], dump_i)
            plsc.store_scatter(ebuf, [idxg], zeros_i)
        dr, pend = lax.fori_loop(0, (kpad - dr) // PW, pair, (dr, pend))
        @pl.when(pend != 0)
        def _():
            wait_scatters()

        pltpu.sync_copy(cntv, c_sh.at[cidx], add=True)
        plsc.subcore_barrier()

        pltpu.sync_copy(a_sh.at[pl.ds(sb * rows_per_tile, rows_per_tile)],
                        a_out.at[r, pl.ds(sb * rows_per_tile, rows_per_tile)])
        pltpu.sync_copy(q_sh.at[pl.ds(sb * rows_per_tile, rows_per_tile)],
                        q_out.at[r, pl.ds(sb * rows_per_tile, rows_per_tile)])
        @pl.when(sb < crows // 8)
        def _():
            pltpu.sync_copy(c_sh.at[pl.ds(sb * 8, 8)],
                            c_out.at[r, pl.ds(sb * 8, 8)])
        if p + 1 < n_rel // NC:
            zero_rows()
            zero_efrows()
            zero_cntv()
            zero_shared()
            plsc.subcore_barrier()


def _final_body(x_ref, a_ref, q_ref, c_ref, wa_ref, wb_ref, we_ref, b_ref,
                o_ref, *, n_rel):
    xv = x_ref[...].astype(jnp.bfloat16)
    av = a_ref[...]
    qv = q_ref[...]
    cnt = c_ref[...]
    wav = wa_ref[...]
    wbv = wb_ref[...]
    wbv16 = wbv.astype(jnp.bfloat16)
    wev = we_ref[...]
    bv = b_ref[...]
    s = 1.0 / jnp.maximum(cnt, 1.0)
    rho = jnp.where(cnt > 0.0, 1.0, 0.0).astype(jnp.float32)
    acc = jnp.zeros(o_ref.shape, jnp.float32)
    for r in range(n_rel):
        gw = jnp.dot(av[r], wbv16[r], preferred_element_type=jnp.float32)
        fw = jnp.dot(qv[r], wev[r], preferred_element_type=jnp.float32)
        zw = jnp.dot(xv, (wav[r] - wbv[r]).astype(jnp.bfloat16),
                     preferred_element_type=jnp.float32)
        acc = acc + s[r][:, None] * (gw + fw) \
                  + rho[r][:, None] * (zw + bv[r][None, :])
    o_ref[...] = acc * (1.0 / n_rel)


def kernel(x, edge_index, edge_type, edge_features, W, b):
    n, d_in = x.shape
    e = edge_index.shape[1]
    n_rel, d_tot, d_out = W.shape
    d_e = d_tot - 2 * d_in
    step = NS * GW
    n_pad = ((n + step - 1) // step) * step

    x_p = jnp.pad(x, ((0, n_pad - n), (0, 0)))
    src = edge_index[0]
    dst = edge_index[1]
    wa = W[:, :d_in, :]
    wb = W[:, d_in:2 * d_in, :]
    we = W[:, 2 * d_in:, :]

    seg = _build_seg(n_pad, e, n_rel, d_in, d_e)
    sh = _spmd_sharding()
    seg_sh = jax.jit(seg, in_shardings=(sh,) * 5, out_shardings=[sh, sh, sh])
    x_sc = x_p.astype(jnp.bfloat16)
    agg, q_agg, cnt2d = seg_sh(x_sc, src, dst, edge_type, edge_features)

    cnt = cnt2d.reshape(n_rel, n_pad)

    bn = 1024
    out_p = pl.pallas_call(
        functools.partial(_final_body, n_rel=n_rel),
        grid=(n_pad // bn,),
        in_specs=[
            pl.BlockSpec((bn, d_in), lambda i: (i, 0)),
            pl.BlockSpec((n_rel, bn, d_in), lambda i: (0, i, 0)),
            pl.BlockSpec((n_rel, bn, d_e), lambda i: (0, i, 0)),
            pl.BlockSpec((n_rel, bn), lambda i: (0, i)),
            pl.BlockSpec((n_rel, d_in, d_out), lambda i: (0, 0, 0)),
            pl.BlockSpec((n_rel, d_in, d_out), lambda i: (0, 0, 0)),
            pl.BlockSpec((n_rel, d_e, d_out), lambda i: (0, 0, 0)),
            pl.BlockSpec((n_rel, d_out), lambda i: (0, 0)),
        ],
        out_specs=pl.BlockSpec((bn, d_out), lambda i: (i, 0)),
        out_shape=jax.ShapeDtypeStruct((n_pad, d_out), jnp.float32),
    )(x_p, agg, q_agg, cnt, wa, wb, we, b)

    return out_p[:n]


def _build_seg(n_pad, e, n_rel, d_in, d_e):
    mesh = plsc.VectorSubcoreMesh(core_axis_name="c", subcore_axis_name="s")
    return pl.kernel(
        functools.partial(_seg_body, n_pad=n_pad, n_edges=e, n_rel=n_rel,
                          d_in=d_in, d_e=d_e),
        out_type=[
            jax.ShapeDtypeStruct((n_rel, n_pad, d_in), jnp.bfloat16),
            jax.ShapeDtypeStruct((n_rel, n_pad, d_e), jnp.float32),
            jax.ShapeDtypeStruct((n_rel, n_pad // 128, 128), jnp.float32),
        ],
        mesh=mesh,
        scratch_types=[
            pltpu.VMEM((2, SB), jnp.int32),
            pltpu.VMEM((2, SB), jnp.int32),
            pltpu.VMEM((2, SB), jnp.int32),
            pltpu.VMEM((RING,), jnp.int32),
            pltpu.VMEM((RING,), jnp.int32),
            pltpu.VMEM((RING,), jnp.int32),
            pltpu.VMEM((GW,), jnp.int32),
            pltpu.VMEM((GW,), jnp.int32),
            pltpu.VMEM((GW,), jnp.int32),
            pltpu.VMEM((GW,), jnp.int32),
            pltpu.VMEM((GW,), jnp.int32),
            pltpu.VMEM((GW,), jnp.int32),
            pltpu.VMEM((n_pad // 128,), jnp.int32),
            pltpu.VMEM((GW, d_in), jnp.bfloat16),
            pltpu.VMEM((GW, d_in), jnp.bfloat16),
            pltpu.VMEM((GW, d_e), jnp.float32),
            pltpu.VMEM((GW, d_e), jnp.float32),
            pltpu.VMEM((n_pad // 128, 128), jnp.float32),
            pltpu.VMEM_SHARED((n_pad, d_in), jnp.bfloat16),
            pltpu.VMEM_SHARED((n_pad, d_e), jnp.float32),
            pltpu.VMEM_SHARED((n_pad // 128, 128), jnp.float32),
            pltpu.SemaphoreType.DMA,
            pltpu.SemaphoreType.DMA,
            pltpu.SemaphoreType.DMA,
            pltpu.SemaphoreType.DMA,
            pltpu.SemaphoreType.DMA,
        ],
        compiler_params=pltpu.CompilerParams(needs_layout_passes=False,
                                             use_tc_tiling_on_sc=False),
    )
